# Initial kernel scaffold; baseline (speedup 1.0000x reference)
import jax
import jax.numpy as jnp
from jax.experimental import pallas as pl


def kernel(x, adj_t, W1, b1, W2, b2):
    raise NotImplementedError("write your pallas kernel here")


# speedup vs baseline: 12.5214x; 12.5214x over previous
import functools

import jax
import jax.numpy as jnp
from jax import lax
from jax.experimental import pallas as pl
from jax.experimental.pallas import tpu as pltpu
from jax.experimental.pallas import tpu_sc as plsc

NC = 2
NS = 16
NT = NC * NS
CHUNK = 128


def _cdiv(a, b):
    return (a + b - 1) // b


def _make_degree(out_rows, k, acc_rows):
    zpt = acc_rows // NS
    rpt = out_rows // NS
    mesh = plsc.VectorSubcoreMesh(core_axis_name="c", subcore_axis_name="s")

    @functools.partial(
        pl.kernel,
        out_type=jax.ShapeDtypeStruct((NC, out_rows, 16), jnp.float32),
        mesh=mesh,
        scratch_types=[
            pltpu.VMEM((k, CHUNK), jnp.int32),
            pltpu.VMEM((CHUNK, 16), jnp.float32),
            pltpu.VMEM_SHARED((acc_rows, 16), jnp.float32),
            pltpu.SemaphoreType.DMA,
        ],
    )
    def deg_kernel(ones_hbm, dst_hbm, zeros_hbm, out_hbm,
                   dst_v, ones_v, acc, sem):
        c = lax.axis_index("c")
        s = lax.axis_index("s")
        w = c * NS + s
        pltpu.sync_copy(zeros_hbm, acc.at[pl.ds(s * zpt, zpt)])
        pltpu.sync_copy(ones_hbm, ones_v)
        pltpu.async_copy(dst_hbm.at[w], dst_v, sem).wait()
        plsc.subcore_barrier()

        @pl.loop(0, k)
        def _(j):
            pltpu.sync_copy(ones_v, acc.at[dst_v.at[j]], add=True)

        plsc.subcore_barrier()
        pltpu.sync_copy(acc.at[pl.ds(s * rpt, rpt)],
                        out_hbm.at[c, pl.ds(s * rpt, rpt)])

    return deg_kernel


def _make_aggregate(out_rows, d, k, acc_rows):
    zpt = acc_rows // NS
    rpt = out_rows // NS
    mesh = plsc.VectorSubcoreMesh(core_axis_name="c", subcore_axis_name="s")

    @functools.partial(
        pl.kernel,
        out_type=jax.ShapeDtypeStruct((NC, out_rows, d), jnp.float32),
        mesh=mesh,
        scratch_types=[
            pltpu.VMEM((k, CHUNK), jnp.int32),
            pltpu.VMEM((k, CHUNK), jnp.int32),
            pltpu.VMEM((CHUNK, d), jnp.float32),
            pltpu.VMEM_SHARED((acc_rows, d), jnp.float32),
            pltpu.SemaphoreType.DMA,
        ],
    )
    def agg_kernel(hp_hbm, src_hbm, dst_hbm, zeros_hbm, out_hbm,
                   src_v, dst_v, rows_v, acc, sem):
        c = lax.axis_index("c")
        s = lax.axis_index("s")
        w = c * NS + s
        pltpu.sync_copy(zeros_hbm, acc.at[pl.ds(s * zpt, zpt)])
        pltpu.async_copy(src_hbm.at[w], src_v, sem).wait()
        pltpu.async_copy(dst_hbm.at[w], dst_v, sem).wait()
        plsc.subcore_barrier()

        @pl.loop(0, k)
        def _(j):
            pltpu.async_copy(hp_hbm.at[src_v.at[j]], rows_v, sem).wait()
            pltpu.sync_copy(rows_v, acc.at[dst_v.at[j]], add=True)

        plsc.subcore_barrier()
        pltpu.sync_copy(acc.at[pl.ds(s * rpt, rpt)],
                        out_hbm.at[c, pl.ds(s * rpt, rpt)])

    return agg_kernel


_ROWS = 1000


def _deg_dis(d0_ref, d1_ref):
    deg = 1.0 + d0_ref[:, :1] + d1_ref[:, :1]
    return deg, lax.rsqrt(deg)


def _tc_prescale(x, w1, d0, d1):
    n, din = x.shape
    d = w1.shape[1]

    def body(x_ref, w_ref, d0_ref, d1_ref, o_ref):
        _, dis = _deg_dis(d0_ref, d1_ref)
        h = jnp.dot(x_ref[...], w_ref[...], preferred_element_type=jnp.float32)
        o_ref[...] = h * dis

    return pl.pallas_call(
        body,
        grid=(n // _ROWS,),
        in_specs=[
            pl.BlockSpec((_ROWS, din), lambda i: (i, 0)),
            pl.BlockSpec((din, d), lambda i: (0, 0)),
            pl.BlockSpec((_ROWS, 16), lambda i: (i, 0)),
            pl.BlockSpec((_ROWS, 16), lambda i: (i, 0)),
        ],
        out_specs=pl.BlockSpec((_ROWS, d), lambda i: (i, 0)),
        out_shape=jax.ShapeDtypeStruct((n, d), jnp.float32),
    )(x, w1, d0, d1)


def _tc_mid(a0, a1, hp1, d0, d1, w2, b1):
    n, d = hp1.shape
    dout = w2.shape[1]

    def body(a0_ref, a1_ref, hp_ref, d0_ref, d1_ref, w_ref, b_ref, o_ref):
        _, dis = _deg_dis(d0_ref, d1_ref)
        o1 = dis * (a0_ref[...] + a1_ref[...] + hp_ref[...]) + b_ref[...]
        o1 = jnp.maximum(o1, 0.0)
        h2 = jnp.dot(o1, w_ref[...], preferred_element_type=jnp.float32)
        o_ref[...] = h2 * dis

    return pl.pallas_call(
        body,
        grid=(n // _ROWS,),
        in_specs=[
            pl.BlockSpec((_ROWS, d), lambda i: (i, 0)),
            pl.BlockSpec((_ROWS, d), lambda i: (i, 0)),
            pl.BlockSpec((_ROWS, d), lambda i: (i, 0)),
            pl.BlockSpec((_ROWS, 16), lambda i: (i, 0)),
            pl.BlockSpec((_ROWS, 16), lambda i: (i, 0)),
            pl.BlockSpec((d, dout), lambda i: (0, 0)),
            pl.BlockSpec((1, d), lambda i: (0, 0)),
        ],
        out_specs=pl.BlockSpec((_ROWS, dout), lambda i: (i, 0)),
        out_shape=jax.ShapeDtypeStruct((n, dout), jnp.float32),
    )(a0, a1, hp1, d0, d1, w2, b1)


def _tc_final(a0, a1, hp2, d0, d1, b2):
    n, d = hp2.shape

    def body(a0_ref, a1_ref, hp_ref, d0_ref, d1_ref, b_ref, o_ref):
        _, dis = _deg_dis(d0_ref, d1_ref)
        o2 = dis * (a0_ref[...] + a1_ref[...] + hp_ref[...]) + b_ref[...]
        o2 = jnp.maximum(o2, 0.0)
        m = jnp.max(o2, axis=1, keepdims=True)
        shifted = o2 - m
        lse = jnp.log(jnp.sum(jnp.exp(shifted), axis=1, keepdims=True))
        o_ref[...] = shifted - lse

    return pl.pallas_call(
        body,
        grid=(n // _ROWS,),
        in_specs=[
            pl.BlockSpec((_ROWS, d), lambda i: (i, 0)),
            pl.BlockSpec((_ROWS, d), lambda i: (i, 0)),
            pl.BlockSpec((_ROWS, d), lambda i: (i, 0)),
            pl.BlockSpec((_ROWS, 16), lambda i: (i, 0)),
            pl.BlockSpec((_ROWS, 16), lambda i: (i, 0)),
            pl.BlockSpec((1, d), lambda i: (0, 0)),
        ],
        out_specs=pl.BlockSpec((_ROWS, d), lambda i: (i, 0)),
        out_shape=jax.ShapeDtypeStruct((n, d), jnp.float32),
    )(a0, a1, hp2, d0, d1, b2)


def kernel(x, adj_t, W1, b1, W2, b2):
    n, din = x.shape
    e = adj_t.shape[1]
    d = W1.shape[1]

    k = _cdiv(e, NT * CHUNK)
    e_pad = NT * k * CHUNK
    out_rows = _cdiv(n, 128) * 128
    acc_rows = out_rows + 128

    src = adj_t[0]
    dst = adj_t[1]
    pad = e_pad - e
    srcp = jnp.concatenate([src, jnp.zeros((pad,), src.dtype)])
    dstp = jnp.concatenate([dst, jnp.full((pad,), n, dst.dtype)])
    src3 = srcp.reshape(NT, k, CHUNK)
    dst3 = dstp.reshape(NT, k, CHUNK)

    zeros16 = jnp.zeros((acc_rows // NS, 16), jnp.float32)
    ones16 = jnp.ones((CHUNK, 16), jnp.float32)
    zeros_d = jnp.zeros((acc_rows // NS, d), jnp.float32)

    degp = _make_degree(out_rows, k, acc_rows)(ones16, dst3, zeros16)
    d0, d1 = degp[0, :n], degp[1, :n]

    agg = _make_aggregate(out_rows, d, k, acc_rows)
    hp1 = _tc_prescale(x, W1, d0, d1)
    agg1 = agg(hp1, src3, dst3, zeros_d)
    hp2 = _tc_mid(agg1[0, :n], agg1[1, :n], hp1, d0, d1, W2, b1.reshape(1, d))
    agg2 = agg(hp2, src3, dst3, zeros_d)
    return _tc_final(agg2[0, :n], agg2[1, :n], hp2, d0, d1, b2.reshape(1, d))
